# baseline (speedup 1.0000x reference)
import functools

import jax
import jax.numpy as jnp
from jax import lax
from jax.experimental import pallas as pl
from jax.experimental.pallas import tpu as pltpu
from jax.experimental.pallas import tpu_sc as plsc

V_DIM = 100000
D = 128
W2 = 20
B = 16384
NC, NS, L = 2, 16, 16
NW = NC * NS
RW = B // NW
C = 8
NBUF = 4
NCHUNK = RW // C
NG = D // L
CW = C * W2
GBLK = [CW]


def _cbow_body(x_ref, y_ref, table_ref, out1_ref, out2_ref,
               idx_x, idx_y, xrows, yrows, out1b, *gsems):
    wid = lax.axis_index("s") * NC + lax.axis_index("c")
    base = wid * RW

    pltpu.sync_copy(x_ref.at[pl.ds(base * W2, RW * W2)], idx_x)
    pltpu.sync_copy(y_ref.at[pl.ds(base, RW)], idx_y)

    def seq_body(i, _):
        idx_x[pl.ds(i * 16, 16)] = (i * 16 + lax.iota(jnp.int32, 16)) & 0xFFFF
        return 0
    lax.fori_loop(0, RW * W2 // 16, seq_body, 0, unroll=False)

    def fire(ch, s):
        off = ch * CW
        o = 0
        for sz in GBLK:
            pltpu.async_copy(table_ref.at[idx_x.at[pl.ds(off + o, sz)]],
                             xrows.at[s, pl.ds(o, sz), :], gsems[s])
            o += sz
        pltpu.async_copy(table_ref.at[idx_y.at[pl.ds(ch * C, C)]],
                         yrows.at[s], gsems[s])

    def drain_gathers(s):
        pltpu.make_async_copy(table_ref.at[pl.ds(0, CW), :],
                              xrows.at[s], gsems[s]).wait()
        pltpu.make_async_copy(table_ref.at[pl.ds(0, C), :],
                              yrows.at[s], gsems[s]).wait()

    def pool(s, row0):
        def pool_body(c, _):
            l0 = c * W2
            accs = [xrows[s, l0, pl.ds(g * L, L)] for g in range(NG)]
            for w in range(1, W2):
                for g in range(NG):
                    accs[g] = accs[g] + xrows[s, l0 + w, pl.ds(g * L, L)]
            for g in range(NG):
                out1b[s, c, pl.ds(g * L, L)] = accs[g] * (1.0 / W2)
            return 0

        lax.fori_loop(0, C, pool_body, 0, unroll=False)
        pltpu.sync_copy(out1b.at[s], out1_ref.at[pl.ds(row0, C), :])
        pltpu.sync_copy(yrows.at[s], out2_ref.at[pl.ds(row0, C), :])

    for s in range(NBUF - 1):
        fire(s, s)

    def ring_body(p, _):
        for s in range(NBUF):
            ch = NBUF * p + s
            nch = ch + NBUF - 1

            @pl.when(nch < NCHUNK)
            def _():
                fire(nch, (s + NBUF - 1) % NBUF)

            drain_gathers(s)
            pool(s, base + ch * C)
        return 0

    lax.fori_loop(0, NCHUNK // NBUF, ring_body, 0, unroll=False)


@jax.jit
def kernel(x, y, table):
    xf = x.reshape(-1)
    mesh = plsc.VectorSubcoreMesh(core_axis_name="c", subcore_axis_name="s",
                                  num_cores=NC, num_subcores=NS)
    out1, out2 = pl.kernel(
        _cbow_body,
        out_type=(jax.ShapeDtypeStruct((B, D), jnp.float32),
                  jax.ShapeDtypeStruct((B, D), jnp.float32)),
        mesh=mesh,
        scratch_types=[
            pltpu.VMEM((RW * W2,), jnp.int32),
            pltpu.VMEM((RW,), jnp.int32),
            pltpu.VMEM((NBUF, CW, D), jnp.float32),
            pltpu.VMEM((NBUF, C, D), jnp.float32),
            pltpu.VMEM((NBUF, C, D), jnp.float32),
        ] + [pltpu.SemaphoreType.DMA] * NBUF,
    )(xf, y, table)
    return (out1, out2)


# speedup vs baseline: 1.3308x; 1.3308x over previous
import functools

import jax
import jax.numpy as jnp
from jax import lax
from jax.experimental import pallas as pl
from jax.experimental.pallas import tpu as pltpu
from jax.experimental.pallas import tpu_sc as plsc

V_DIM = 100000
D = 128
W2 = 20
B = 16384
NC, NS, L = 2, 16, 16
NW = NC * NS
RW = B // NW
C = 8
NBUF = 2
NCHUNK = RW // C
NG = D // L
CW = C * W2
GBLK = [CW]


def _cbow_body(x_ref, y_ref, table_ref, out1_ref, out2_ref,
               idx_x, idx_y, xrows, yrows, out1b, *gsems):
    wid = lax.axis_index("s") * NC + lax.axis_index("c")
    base = wid * RW

    pltpu.sync_copy(x_ref.at[pl.ds(base, RW), :], idx_x)
    pltpu.sync_copy(y_ref.at[pl.ds(base, RW)], idx_y)

    def fire(ch, s):
        for c in range(C):
            pltpu.async_copy(table_ref.at[idx_x.at[ch * C + c]],
                             xrows.at[s, pl.ds(c * W2, W2), :], gsems[s])
        pltpu.async_copy(table_ref.at[idx_y.at[pl.ds(ch * C, C)]],
                         yrows.at[s], gsems[s])

    def drain_gathers(s):
        pltpu.make_async_copy(table_ref.at[pl.ds(0, CW), :],
                              xrows.at[s], gsems[s]).wait()
        pltpu.make_async_copy(table_ref.at[pl.ds(0, C), :],
                              yrows.at[s], gsems[s]).wait()

    def pool(s, row0):
        def pool_body(c, _):
            l0 = c * W2
            accs = [xrows[s, l0, pl.ds(g * L, L)] for g in range(NG)]
            for w in range(1, W2):
                for g in range(NG):
                    accs[g] = accs[g] + xrows[s, l0 + w, pl.ds(g * L, L)]
            for g in range(NG):
                out1b[s, c, pl.ds(g * L, L)] = accs[g] * (1.0 / W2)
            return 0

        lax.fori_loop(0, C, pool_body, 0, unroll=False)
        pltpu.sync_copy(out1b.at[s], out1_ref.at[pl.ds(row0, C), :])
        pltpu.sync_copy(yrows.at[s], out2_ref.at[pl.ds(row0, C), :])

    for s in range(NBUF - 1):
        fire(s, s)

    def ring_body(p, _):
        for s in range(NBUF):
            ch = NBUF * p + s
            nch = ch + NBUF - 1

            @pl.when(nch < NCHUNK)
            def _():
                fire(nch, (s + NBUF - 1) % NBUF)

            drain_gathers(s)
            pool(s, base + ch * C)
        return 0

    lax.fori_loop(0, NCHUNK // NBUF, ring_body, 0, unroll=False)


@jax.jit
def kernel(x, y, table):
    mesh = plsc.VectorSubcoreMesh(core_axis_name="c", subcore_axis_name="s",
                                  num_cores=NC, num_subcores=NS)
    out1, out2 = pl.kernel(
        _cbow_body,
        out_type=(jax.ShapeDtypeStruct((B, D), jnp.float32),
                  jax.ShapeDtypeStruct((B, D), jnp.float32)),
        mesh=mesh,
        scratch_types=[
            pltpu.VMEM((RW, W2), jnp.int32),
            pltpu.VMEM((RW,), jnp.int32),
            pltpu.VMEM((NBUF, CW, D), jnp.float32),
            pltpu.VMEM((NBUF, C, D), jnp.float32),
            pltpu.VMEM((NBUF, C, D), jnp.float32),
        ] + [pltpu.SemaphoreType.DMA] * NBUF,
    )(x, y, table)
    return (out1, out2)


# speedup vs baseline: 1.5587x; 1.1713x over previous
import functools

import jax
import jax.numpy as jnp
from jax import lax
from jax.experimental import pallas as pl
from jax.experimental.pallas import tpu as pltpu
from jax.experimental.pallas import tpu_sc as plsc

V_DIM = 100000
D = 128
W2 = 20
B = 16384
NC, NS, L = 2, 16, 16
NW = NC * NS
RW = B // NW
C = 8
NBUF = 4
NCHUNK = RW // C
NG = D // L
CW = C * W2


def _cbow_body(x_ref, y_ref, table_ref, out1_ref, out2_ref,
               idxc, idx_y, xrows, yrows, out1b, isems, gsems):
    wid = lax.axis_index("s") * NC + lax.axis_index("c")
    base = wid * RW

    pltpu.sync_copy(y_ref.at[pl.ds(base, RW)], idx_y)

    def stage(ch, s):
        pltpu.async_copy(x_ref.at[pl.ds(base + ch * C, C), :],
                         idxc.at[s], isems[s])

    def fire(ch, s):
        pltpu.make_async_copy(x_ref.at[pl.ds(0, C), :],
                              idxc.at[s], isems[s]).wait()
        for c in range(C):
            pltpu.async_copy(table_ref.at[idxc.at[s, c]],
                             xrows.at[s, pl.ds(c * W2, W2), :], gsems[s])
        pltpu.async_copy(table_ref.at[idx_y.at[pl.ds(ch * C, C)]],
                         yrows.at[s], gsems[s])

    def drain_gathers(s):
        pltpu.make_async_copy(table_ref.at[pl.ds(0, CW), :],
                              xrows.at[s], gsems[s]).wait()
        pltpu.make_async_copy(table_ref.at[pl.ds(0, C), :],
                              yrows.at[s], gsems[s]).wait()

    def pool(s, row0):
        def pool_body(c, _):
            l0 = c * W2
            accs = [xrows[s, l0, pl.ds(g * L, L)] for g in range(NG)]
            for w in range(1, W2):
                for g in range(NG):
                    accs[g] = accs[g] + xrows[s, l0 + w, pl.ds(g * L, L)]
            for g in range(NG):
                out1b[s, c, pl.ds(g * L, L)] = accs[g] * (1.0 / W2)
            return 0

        lax.fori_loop(0, C, pool_body, 0, unroll=False)
        pltpu.sync_copy(out1b.at[s], out1_ref.at[pl.ds(row0, C), :])
        pltpu.sync_copy(yrows.at[s], out2_ref.at[pl.ds(row0, C), :])

    for s in range(NBUF):
        stage(s, s)
    for s in range(NBUF - 1):
        fire(s, s)

    def ring_body(p, _):
        for s in range(NBUF):
            ch = NBUF * p + s
            nch = ch + NBUF - 1

            @pl.when(nch < NCHUNK)
            def _():
                fire(nch, (s + NBUF - 1) % NBUF)

            @pl.when(nch + 1 < NCHUNK)
            def _():
                stage(nch + 1, s)

            drain_gathers(s)
            pool(s, base + ch * C)
        return 0

    lax.fori_loop(0, NCHUNK // NBUF, ring_body, 0, unroll=False)


@jax.jit
def kernel(x, y, table):
    mesh = plsc.VectorSubcoreMesh(core_axis_name="c", subcore_axis_name="s",
                                  num_cores=NC, num_subcores=NS)
    out1, out2 = pl.kernel(
        _cbow_body,
        out_type=(jax.ShapeDtypeStruct((B, D), jnp.float32),
                  jax.ShapeDtypeStruct((B, D), jnp.float32)),
        mesh=mesh,
        scratch_types=[
            pltpu.VMEM((NBUF, C, W2), jnp.int32),
            pltpu.VMEM((RW,), jnp.int32),
            pltpu.VMEM((NBUF, CW, D), jnp.float32),
            pltpu.VMEM((NBUF, C, D), jnp.float32),
            pltpu.VMEM((NBUF, C, D), jnp.float32),
            [pltpu.SemaphoreType.DMA] * NBUF,
            [pltpu.SemaphoreType.DMA] * NBUF,
        ],
    )(x, y, table)
    return (out1, out2)


# speedup vs baseline: 1.5587x; 1.0000x over previous
import functools

import jax
import jax.numpy as jnp
from jax import lax
from jax.experimental import pallas as pl
from jax.experimental.pallas import tpu as pltpu
from jax.experimental.pallas import tpu_sc as plsc

V_DIM = 100000
D = 128
W2 = 20
B = 16384
NC, NS, L = 2, 16, 16
NW = NC * NS
RW = B // NW
C = 8
NBUF = 4
NCHUNK = RW // C
NG = D // L
CW = C * W2


def _cbow_body(x_ref, y_ref, table_ref, out1_ref, out2_ref,
               idxc, idx_y, xrows, yrows, out1b, isems, gsems):
    wid = lax.axis_index("s") * NC + lax.axis_index("c")
    base = wid * RW

    pltpu.sync_copy(y_ref.at[pl.ds(base, RW)], idx_y)

    def stage(ch, s):
        pltpu.async_copy(x_ref.at[pl.ds(base + ch * C, C), :],
                         idxc.at[s], isems[s])

    def fire(ch, s):
        pltpu.make_async_copy(x_ref.at[pl.ds(0, C), :],
                              idxc.at[s], isems[s]).wait()
        for c in range(C):
            pltpu.async_copy(table_ref.at[idxc.at[s, c]],
                             xrows.at[s, pl.ds(c * W2, W2), :], gsems[s])
        pltpu.async_copy(table_ref.at[idx_y.at[pl.ds(ch * C, C)]],
                         yrows.at[s], gsems[s])

    def drain_gathers(s):
        pltpu.make_async_copy(table_ref.at[pl.ds(0, CW), :],
                              xrows.at[s], gsems[s]).wait()
        pltpu.make_async_copy(table_ref.at[pl.ds(0, C), :],
                              yrows.at[s], gsems[s]).wait()

    def pool(s, row0):
        def pool_body(c, _):
            l0 = c * W2
            accs = [xrows[s, l0, pl.ds(g * L, L)] for g in range(NG)]
            for w in range(1, W2):
                for g in range(NG):
                    accs[g] = accs[g] + xrows[s, l0 + w, pl.ds(g * L, L)]
            for g in range(NG):
                out1b[s, c, pl.ds(g * L, L)] = accs[g] * (1.0 / W2)
            return 0

        lax.fori_loop(0, C, pool_body, 0, unroll=False)
        pltpu.sync_copy(out1b.at[s], out1_ref.at[pl.ds(row0, C), :])
        pltpu.sync_copy(yrows.at[s], out2_ref.at[pl.ds(row0, C), :])

    for s in range(NBUF):
        stage(s, s)
    for s in range(NBUF - 1):
        fire(s, s)

    def ring_body(p, _):
        for s in range(NBUF):
            ch = NBUF * p + s
            nch = ch + NBUF - 1

            @pl.when(nch < NCHUNK)
            def _():
                fire(nch, (s + NBUF - 1) % NBUF)

            @pl.when(nch + 1 < NCHUNK)
            def _():
                stage(nch + 1, s)

            drain_gathers(s)
            pool(s, base + ch * C)
        return 0

    lax.fori_loop(0, NCHUNK // NBUF, ring_body, 0, unroll=False)


@jax.jit
def kernel(x, y, table):
    mesh = plsc.VectorSubcoreMesh(core_axis_name="c", subcore_axis_name="s",
                                  num_cores=NC, num_subcores=NS)
    out1, out2 = pl.kernel(
        _cbow_body,
        out_type=(jax.ShapeDtypeStruct((B, D), jnp.float32),
                  jax.ShapeDtypeStruct((B, D), jnp.float32)),
        mesh=mesh,
        compiler_params=pltpu.CompilerParams(use_tc_tiling_on_sc=True),
        scratch_types=[
            pltpu.VMEM((NBUF, C, W2), jnp.int32),
            pltpu.VMEM((RW,), jnp.int32),
            pltpu.VMEM((NBUF, CW, D), jnp.float32),
            pltpu.VMEM((NBUF, C, D), jnp.float32),
            pltpu.VMEM((NBUF, C, D), jnp.float32),
            [pltpu.SemaphoreType.DMA] * NBUF,
            [pltpu.SemaphoreType.DMA] * NBUF,
        ],
    )(x, y, table)
    return (out1, out2)


# speedup vs baseline: 1.5999x; 1.0264x over previous
import functools

import jax
import jax.numpy as jnp
from jax import lax
from jax.experimental import pallas as pl
from jax.experimental.pallas import tpu as pltpu
from jax.experimental.pallas import tpu_sc as plsc

V_DIM = 100000
D = 128
W2 = 20
B = 16384
NC, NS, L = 2, 16, 16
NW = NC * NS
RW = B // NW
C = 8
NBUF = 4
NCHUNK = RW // C
NG = D // L
CW = C * W2


def _cbow_body(x_ref, y_ref, table_ref, out1_ref, out2_ref,
               idxc, idx_y, xrows, yrows, out1b, isems, gsems, wsems):
    wid = lax.axis_index("s") * NC + lax.axis_index("c")
    base = wid * RW

    pltpu.sync_copy(y_ref.at[pl.ds(base, RW)], idx_y)

    def stage(ch, s):
        pltpu.async_copy(x_ref.at[pl.ds(base + ch * C, C), :],
                         idxc.at[s], isems[s])

    def drain_writes(s):
        pltpu.make_async_copy(out1b.at[s], out1_ref.at[pl.ds(0, C), :],
                              wsems[s]).wait()
        pltpu.make_async_copy(yrows.at[s], out2_ref.at[pl.ds(0, C), :],
                              wsems[s]).wait()

    def fire(ch, s):
        pltpu.make_async_copy(x_ref.at[pl.ds(0, C), :],
                              idxc.at[s], isems[s]).wait()
        for c in range(C):
            pltpu.async_copy(table_ref.at[idxc.at[s, c]],
                             xrows.at[s, pl.ds(c * W2, W2), :], gsems[s])
        pltpu.async_copy(table_ref.at[idx_y.at[pl.ds(ch * C, C)]],
                         yrows.at[s], gsems[s])

    def drain_gathers(s):
        pltpu.make_async_copy(table_ref.at[pl.ds(0, CW), :],
                              xrows.at[s], gsems[s]).wait()
        pltpu.make_async_copy(table_ref.at[pl.ds(0, C), :],
                              yrows.at[s], gsems[s]).wait()

    def pool(s, row0):
        def pool_body(c, _):
            l0 = c * W2
            accs = [xrows[s, l0, pl.ds(g * L, L)] for g in range(NG)]
            for w in range(1, W2):
                for g in range(NG):
                    accs[g] = accs[g] + xrows[s, l0 + w, pl.ds(g * L, L)]
            for g in range(NG):
                out1b[s, c, pl.ds(g * L, L)] = accs[g] * (1.0 / W2)
            return 0

        lax.fori_loop(0, C, pool_body, 0, unroll=False)
        pltpu.async_copy(out1b.at[s], out1_ref.at[pl.ds(row0, C), :],
                         wsems[s])
        pltpu.async_copy(yrows.at[s], out2_ref.at[pl.ds(row0, C), :],
                         wsems[s])

    for s in range(NBUF):
        stage(s, s)
    for s in range(NBUF - 1):
        fire(s, s)

    def ring_body(p, _):
        for s in range(NBUF):
            ch = NBUF * p + s
            nch = ch + NBUF - 1

            @pl.when(nch < NCHUNK)
            def _():
                @pl.when(ch >= 1)
                def _():
                    drain_writes((s + NBUF - 1) % NBUF)
                fire(nch, (s + NBUF - 1) % NBUF)

            @pl.when(nch + 1 < NCHUNK)
            def _():
                stage(nch + 1, s)

            drain_gathers(s)
            pool(s, base + ch * C)
        return 0

    lax.fori_loop(0, NCHUNK // NBUF, ring_body, 0, unroll=False)
    for s in range(NBUF):
        drain_writes(s)


@jax.jit
def kernel(x, y, table):
    mesh = plsc.VectorSubcoreMesh(core_axis_name="c", subcore_axis_name="s",
                                  num_cores=NC, num_subcores=NS)
    out1, out2 = pl.kernel(
        _cbow_body,
        out_type=(jax.ShapeDtypeStruct((B, D), jnp.float32),
                  jax.ShapeDtypeStruct((B, D), jnp.float32)),
        mesh=mesh,
        scratch_types=[
            pltpu.VMEM((NBUF, C, W2), jnp.int32),
            pltpu.VMEM((RW,), jnp.int32),
            pltpu.VMEM((NBUF, CW, D), jnp.float32),
            pltpu.VMEM((NBUF, C, D), jnp.float32),
            pltpu.VMEM((NBUF, C, D), jnp.float32),
            [pltpu.SemaphoreType.DMA] * NBUF,
            [pltpu.SemaphoreType.DMA] * NBUF,
            [pltpu.SemaphoreType.DMA] * NBUF,
        ],
    )(x, y, table)
    return (out1, out2)


# speedup vs baseline: 1.6037x; 1.0024x over previous
import functools

import jax
import jax.numpy as jnp
from jax import lax
from jax.experimental import pallas as pl
from jax.experimental.pallas import tpu as pltpu
from jax.experimental.pallas import tpu_sc as plsc

V_DIM = 100000
D = 128
W2 = 20
B = 16384
NC, NS, L = 2, 16, 16
NW = NC * NS
RW = B // NW
C = 8
NBUF = 4
NCHUNK = RW // C
NG = D // L
CW = C * W2


def _cbow_body(x_ref, y_ref, table_ref, out1_ref, out2_ref,
               idxc, idx_y, xrows, yrows, out1b, isems, gsems, wsems):
    wid = lax.axis_index("s") * NC + lax.axis_index("c")
    base = wid * RW

    pltpu.sync_copy(y_ref.at[pl.ds(base, RW)], idx_y)

    def stage(ch, s):
        pltpu.async_copy(x_ref.at[pl.ds(base + ch * C, C), :],
                         idxc.at[s], isems[s])

    def drain_writes(s):
        pltpu.make_async_copy(out1b.at[s], out1_ref.at[pl.ds(0, C), :],
                              wsems[s]).wait()
        pltpu.make_async_copy(yrows.at[s], out2_ref.at[pl.ds(0, C), :],
                              wsems[s]).wait()

    def fire(ch, s):
        pltpu.make_async_copy(x_ref.at[pl.ds(0, C), :],
                              idxc.at[s], isems[s]).wait()
        for c in range(C):
            pltpu.async_copy(table_ref.at[idxc.at[s, c]],
                             xrows.at[s, pl.ds(c * W2, W2), :], gsems[s])
        pltpu.async_copy(table_ref.at[idx_y.at[pl.ds(ch * C, C)]],
                         yrows.at[s], gsems[s])

    def drain_gathers(s):
        pltpu.make_async_copy(table_ref.at[pl.ds(0, CW), :],
                              xrows.at[s], gsems[s]).wait()
        pltpu.make_async_copy(table_ref.at[pl.ds(0, C), :],
                              yrows.at[s], gsems[s]).wait()

    def pool(s, row0):
        def pool_body(c, _):
            l0 = c * W2
            accs = [xrows[s, l0, pl.ds(g * L, L)] for g in range(NG)]
            for w in range(1, W2):
                for g in range(NG):
                    accs[g] = accs[g] + xrows[s, l0 + w, pl.ds(g * L, L)]
            for g in range(NG):
                out1b[s, c, pl.ds(g * L, L)] = accs[g] * (1.0 / W2)
            return 0

        lax.fori_loop(0, C, pool_body, 0, unroll=False)
        pltpu.async_copy(out1b.at[s], out1_ref.at[pl.ds(row0, C), :],
                         wsems[s])
        pltpu.async_copy(yrows.at[s], out2_ref.at[pl.ds(row0, C), :],
                         wsems[s])

    for s in range(NBUF):
        stage(s, s)
    for s in range(NBUF - 1):
        fire(s, s)

    def ring_body(p, _):
        for s in range(NBUF):
            ch = NBUF * p + s
            nch = ch + NBUF - 1

            @pl.when(nch < NCHUNK)
            def _():
                @pl.when(ch >= 1)
                def _():
                    drain_writes((s + NBUF - 1) % NBUF)
                fire(nch, (s + NBUF - 1) % NBUF)

            @pl.when(nch + 1 < NCHUNK)
            def _():
                stage(nch + 1, s)

            drain_gathers(s)
            pool(s, base + ch * C)
        return 0

    lax.fori_loop(0, NCHUNK // NBUF, ring_body, 0, unroll=False)
    for s in range(NBUF):
        drain_writes(s)


@jax.jit
def kernel(x, y, table):
    mesh = plsc.VectorSubcoreMesh(core_axis_name="c", subcore_axis_name="s",
                                  num_cores=NC, num_subcores=NS)
    out1, out2 = pl.kernel(
        _cbow_body,
        out_type=(jax.ShapeDtypeStruct((B, D), jnp.float32),
                  jax.ShapeDtypeStruct((B, D), jnp.float32)),
        mesh=mesh,
        compiler_params=pltpu.CompilerParams(use_tc_tiling_on_sc=True),
        scratch_types=[
            pltpu.VMEM((NBUF, C, W2), jnp.int32),
            pltpu.VMEM((RW,), jnp.int32),
            pltpu.VMEM((NBUF, CW, D), jnp.float32),
            pltpu.VMEM((NBUF, C, D), jnp.float32),
            pltpu.VMEM((NBUF, C, D), jnp.float32),
            [pltpu.SemaphoreType.DMA] * NBUF,
            [pltpu.SemaphoreType.DMA] * NBUF,
            [pltpu.SemaphoreType.DMA] * NBUF,
        ],
    )(x, y, table)
    return (out1, out2)
